# Initial kernel scaffold; baseline (speedup 1.0000x reference)
import functools

import jax
import jax.numpy as jnp
from jax import lax
from jax.experimental import pallas as pl
from jax.experimental.pallas import tpu as pltpu
from jax.experimental.pallas import tpu_sc as plsc

N = 10000
E = 320000
D = 128
H = 256
O = 64

NLANE = 16
NCORE = 2
NSUB = 16
NW = NCORE * NSUB
C = 128
K = 82
EPAD = NW * K * C
TOT_CHUNKS = EPAD // C
NACC = 10240
ROWS_PER_TILE = NACC // NSUB
PADN = 10016
DUMMY = N


def _tc_head(x, W, att_src, att_dst):
    def body(x_ref, w_ref, as_ref, ad_ref, h_ref, s_ref, d_ref, c_ref):
        h = jnp.dot(x_ref[...], w_ref[...], preferred_element_type=jnp.float32)
        h_ref[...] = h
        a_s = jnp.sum(h * as_ref[...][None, :], axis=1, keepdims=True)
        a_d = jnp.sum(h * ad_ref[...][None, :], axis=1, keepdims=True)
        s_ref[...] = a_s
        d_ref[...] = a_d
        c = jnp.maximum(jnp.max(a_s) + jnp.max(a_d), 0.0)
        c_ref[...] = jnp.full((1, NLANE), c, jnp.float32)

    return pl.pallas_call(
        body,
        out_shape=(
            jax.ShapeDtypeStruct((N, D), jnp.float32),
            jax.ShapeDtypeStruct((N, 1), jnp.float32),
            jax.ShapeDtypeStruct((N, 1), jnp.float32),
            jax.ShapeDtypeStruct((1, NLANE), jnp.float32),
        ),
    )(x, W, att_src, att_dst)


def _sc_gat(h, packed_idx, a_s, a_d, cvec):
    mesh = plsc.VectorSubcoreMesh(core_axis_name="c", subcore_axis_name="s")

    @functools.partial(
        pl.kernel,
        out_type=(
            jax.ShapeDtypeStruct((NCORE, NACC, D), jnp.float32),
            jax.ShapeDtypeStruct((NCORE, NACC, NLANE), jnp.float32),
        ),
        mesh=mesh,
        scratch_types=[
            pltpu.VMEM((PADN,), jnp.float32),
            pltpu.VMEM((PADN,), jnp.float32),
            pltpu.VMEM((NLANE,), jnp.float32),
            pltpu.VMEM((2, C), jnp.int32),
            pltpu.VMEM((C, D), jnp.float32),
            pltpu.VMEM((C,), jnp.float32),
            pltpu.VMEM((C, NLANE), jnp.float32),
            pltpu.VMEM_SHARED((NACC, D), jnp.float32),
            pltpu.VMEM_SHARED((NACC, NLANE), jnp.float32),
            pltpu.SemaphoreType.DMA,
        ],
    )
    def kern(h_hbm, idx_hbm, as_hbm, ad_hbm, c_hbm, acc_out, accp_out,
             as_v, ad_v, c_v, idx_v, rows_v, p_v, pden_v, acc_sh, accp_sh,
             sem):
        cid = lax.axis_index("c")
        sid = lax.axis_index("s")
        wid = sid * NCORE + cid

        pltpu.sync_copy(as_hbm, as_v)
        pltpu.sync_copy(ad_hbm, ad_v)
        pltpu.sync_copy(c_hbm, c_v)
        shift = c_v[0]

        zv = jnp.zeros((NLANE,), jnp.float32)

        @pl.loop(0, C)
        def _(j):
            for q in range(D // NLANE):
                rows_v[j, pl.ds(q * NLANE, NLANE)] = zv
            pden_v[j, :] = zv

        zbase = sid * ROWS_PER_TILE
        for t in range(ROWS_PER_TILE // C):
            pltpu.sync_copy(rows_v, acc_sh.at[pl.ds(zbase + t * C, C)])
            pltpu.sync_copy(pden_v, accp_sh.at[pl.ds(zbase + t * C, C)])
        plsc.subcore_barrier()

        chunk0 = wid * K

        @pl.loop(0, K)
        def _(k):
            ck = chunk0 + k
            pltpu.sync_copy(idx_hbm.at[ck], idx_v)
            pltpu.async_copy(h_hbm.at[idx_v.at[0]], rows_v, sem).wait()
            for b in range(C // NLANE):
                sl = pl.ds(b * NLANE, NLANE)
                av = plsc.load_gather(as_v, [idx_v[0, sl]])
                bv = plsc.load_gather(ad_v, [idx_v[1, sl]])
                e = av + bv
                e = jnp.maximum(e, e * 0.2)
                p_v[sl] = jnp.exp(e - shift)

            @pl.loop(0, C)
            def _(j):
                pj = p_v[j]
                pvec = jnp.full((NLANE,), pj, jnp.float32)
                pden_v[j, :] = pvec
                for q in range(D // NLANE):
                    sl = pl.ds(q * NLANE, NLANE)
                    rows_v[j, sl] = rows_v[j, sl] * pvec

            pltpu.sync_copy(rows_v, acc_sh.at[idx_v.at[1]], add=True)
            pltpu.sync_copy(pden_v, accp_sh.at[idx_v.at[1]], add=True)

        plsc.subcore_barrier()
        pltpu.sync_copy(acc_sh.at[pl.ds(zbase, ROWS_PER_TILE)],
                        acc_out.at[cid, pl.ds(zbase, ROWS_PER_TILE)])
        pltpu.sync_copy(accp_sh.at[pl.ds(zbase, ROWS_PER_TILE)],
                        accp_out.at[cid, pl.ds(zbase, ROWS_PER_TILE)])

    return kern(h, packed_idx, a_s, a_d, cvec)


def _tc_mlp(acc, accp, bias, W1, b1, W2, b2):
    BR = 1024

    def body(acc_ref, accp_ref, bias_ref, w1_ref, b1_ref, w2_ref, b2_ref,
             y_ref):
        g = acc_ref[0] + acc_ref[1]
        den = accp_ref[0, :, 0:1] + accp_ref[1, :, 0:1]
        gat = g / den + bias_ref[...][None, :]
        z = jnp.dot(gat, w1_ref[...], preferred_element_type=jnp.float32)
        z = jnp.maximum(z + b1_ref[...][None, :], 0.0)
        y = jnp.dot(z, w2_ref[...], preferred_element_type=jnp.float32)
        y_ref[...] = jax.nn.sigmoid(y + b2_ref[...][None, :])

    return pl.pallas_call(
        body,
        grid=(NACC // BR,),
        in_specs=[
            pl.BlockSpec((NCORE, BR, D), lambda i: (0, i, 0)),
            pl.BlockSpec((NCORE, BR, NLANE), lambda i: (0, i, 0)),
            pl.BlockSpec((D,), lambda i: (0,)),
            pl.BlockSpec((D, H), lambda i: (0, 0)),
            pl.BlockSpec((H,), lambda i: (0,)),
            pl.BlockSpec((H, O), lambda i: (0, 0)),
            pl.BlockSpec((O,), lambda i: (0,)),
        ],
        out_specs=pl.BlockSpec((BR, O), lambda i: (i, 0)),
        out_shape=jax.ShapeDtypeStruct((NACC, O), jnp.float32),
    )(acc, accp, bias, W1, b1, W2, b2)


def kernel(x, edge_index, W, att_src, att_dst, bias, W1, b1, W2, b2):
    h, a_s2, a_d2, cvec = _tc_head(x, W, att_src, att_dst)

    loop = jnp.arange(N, dtype=jnp.int32)
    npad = EPAD - E - N
    src = jnp.concatenate([edge_index[0], loop,
                           jnp.zeros((npad,), jnp.int32)])
    dst = jnp.concatenate([edge_index[1], loop,
                           jnp.full((npad,), DUMMY, jnp.int32)])
    packed = jnp.stack([src.reshape(TOT_CHUNKS, C),
                        dst.reshape(TOT_CHUNKS, C)], axis=1)

    a_s = jnp.pad(a_s2.reshape(-1), (0, PADN - N))
    a_d = jnp.pad(a_d2.reshape(-1), (0, PADN - N))

    acc, accp = _sc_gat(h, packed, a_s, a_d, cvec.reshape(-1))
    y = _tc_mlp(acc, accp, bias, W1, b1, W2, b2)
    return y[:N]


# speedup vs baseline: 16.2369x; 16.2369x over previous
import dataclasses
import functools

import jax
import jax.numpy as jnp
from jax import lax
from jax.experimental import pallas as pl
from jax.experimental.pallas import tpu as pltpu
from jax.experimental.pallas import tpu_sc as plsc

N = 10000
E = 320000
D = 128
H = 256
O = 64

NLANE = 16
NCORE = 2
NSUB = 16
NW = NCORE * NSUB
C = 128
K = 82
EPAD = NW * K * C
TOT_CHUNKS = EPAD // C
NACC = 10240
ROWS_PER_TILE = NACC // NSUB
PADN = 10016
DUMMY = N


def _tc_head(x, W, att_src, att_dst):
    def body(x_ref, w_ref, as_ref, ad_ref, h_ref, s_ref, d_ref, c_ref):
        h = jnp.dot(x_ref[...], w_ref[...], preferred_element_type=jnp.float32)
        h_ref[...] = h
        a_s = jnp.sum(h * as_ref[...][None, :], axis=1, keepdims=True)
        a_d = jnp.sum(h * ad_ref[...][None, :], axis=1, keepdims=True)
        s_ref[...] = a_s
        d_ref[...] = a_d
        c = jnp.maximum(jnp.max(a_s) + jnp.max(a_d), 0.0)
        c_ref[...] = jnp.full((1, NLANE), c, jnp.float32)

    return pl.pallas_call(
        body,
        out_shape=(
            jax.ShapeDtypeStruct((N, D), jnp.float32),
            jax.ShapeDtypeStruct((N, 1), jnp.float32),
            jax.ShapeDtypeStruct((N, 1), jnp.float32),
            jax.ShapeDtypeStruct((1, NLANE), jnp.float32),
        ),
    )(x, W, att_src, att_dst)


def _sc_gat(h, src_idx, dst_idx, a_s, a_d, cvec):
    mesh = plsc.VectorSubcoreMesh(core_axis_name="c", subcore_axis_name="s")
    cp = pltpu.CompilerParams()
    if "needs_layout_passes" in pltpu.CompilerParams.__dataclass_fields__:
        cp = dataclasses.replace(cp, needs_layout_passes=False)

    @functools.partial(
        pl.kernel,
        compiler_params=cp,
        out_type=(
            jax.ShapeDtypeStruct((NCORE, NACC, D), jnp.float32),
            jax.ShapeDtypeStruct((NW, NACC), jnp.float32),
        ),
        mesh=mesh,
        scratch_types=[
            pltpu.VMEM((PADN,), jnp.float32),
            pltpu.VMEM((PADN,), jnp.float32),
            pltpu.VMEM((NLANE,), jnp.float32),
            pltpu.VMEM((C,), jnp.int32),
            pltpu.VMEM((C,), jnp.int32),
            pltpu.VMEM((C, D), jnp.float32),
            pltpu.VMEM((C,), jnp.float32),
            pltpu.VMEM((NACC,), jnp.float32),
            pltpu.VMEM_SHARED((NACC, D), jnp.float32),
            pltpu.SemaphoreType.DMA,
        ],
    )
    def kern(h_hbm, src_hbm, dst_hbm, as_hbm, ad_hbm, c_hbm, acc_out, den_out,
             as_v, ad_v, c_v, src_v, dst_v, rows_v, p_v, den_v, acc_sh, sem):
        cid = lax.axis_index("c")
        sid = lax.axis_index("s")
        wid = sid * NCORE + cid

        pltpu.sync_copy(as_hbm, as_v)
        pltpu.sync_copy(ad_hbm, ad_v)
        pltpu.sync_copy(c_hbm, c_v)
        shift = c_v[...]

        zv = jnp.zeros((NLANE,), jnp.float32)

        @pl.loop(0, NACC, step=NLANE)
        def _(i):
            den_v[pl.ds(i, NLANE)] = zv

        @pl.loop(0, C)
        def _(j):
            for q in range(D // NLANE):
                rows_v[j, pl.ds(q * NLANE, NLANE)] = zv

        zbase = sid * ROWS_PER_TILE
        for t in range(ROWS_PER_TILE // C):
            pltpu.sync_copy(rows_v, acc_sh.at[pl.ds(zbase + t * C, C)])
        plsc.subcore_barrier()

        chunk0 = wid * K

        @pl.loop(0, K)
        def _(k):
            ck = chunk0 + k
            pltpu.sync_copy(src_hbm.at[ck], src_v)
            pltpu.sync_copy(dst_hbm.at[ck], dst_v)
            pltpu.async_copy(h_hbm.at[src_v], rows_v, sem).wait()
            for b in range(C // NLANE):
                sl = pl.ds(b * NLANE, NLANE)
                d16 = dst_v[sl]
                av = plsc.load_gather(as_v, [src_v[sl]])
                bv = plsc.load_gather(ad_v, [d16])
                e = av + bv
                e = jnp.maximum(e, e * 0.2)
                p16 = jnp.exp(e - shift)
                p_v[sl] = p16
                plsc.addupdate_scatter(den_v, [d16], p16)

            @pl.loop(0, C)
            def _(j):
                jv = jnp.full((NLANE,), j, jnp.int32)
                pvec = plsc.load_gather(p_v, [jv])
                for q in range(D // NLANE):
                    sl = pl.ds(q * NLANE, NLANE)
                    rows_v[j, sl] = rows_v[j, sl] * pvec

            pltpu.sync_copy(rows_v, acc_sh.at[dst_v], add=True)

        plsc.subcore_barrier()
        pltpu.sync_copy(acc_sh.at[pl.ds(zbase, ROWS_PER_TILE)],
                        acc_out.at[cid, pl.ds(zbase, ROWS_PER_TILE)])
        pltpu.sync_copy(den_v, den_out.at[wid])

    return kern(h, src_idx, dst_idx, a_s, a_d, cvec)


def _tc_mlp(acc, den, bias, W1, b1, W2, b2):
    BR = 1024

    def body(acc_ref, den_ref, bias_ref, w1_ref, b1_ref, w2_ref, b2_ref,
             y_ref):
        g = acc_ref[0] + acc_ref[1]
        d = jnp.sum(den_ref[...], axis=0).reshape(BR, 1)
        gat = g / d + bias_ref[...][None, :]
        z = jnp.dot(gat, w1_ref[...], preferred_element_type=jnp.float32)
        z = jnp.maximum(z + b1_ref[...][None, :], 0.0)
        y = jnp.dot(z, w2_ref[...], preferred_element_type=jnp.float32)
        y_ref[...] = jax.nn.sigmoid(y + b2_ref[...][None, :])

    return pl.pallas_call(
        body,
        grid=(NACC // BR,),
        in_specs=[
            pl.BlockSpec((NCORE, BR, D), lambda i: (0, i, 0)),
            pl.BlockSpec((NW, BR), lambda i: (0, i)),
            pl.BlockSpec((D,), lambda i: (0,)),
            pl.BlockSpec((D, H), lambda i: (0, 0)),
            pl.BlockSpec((H,), lambda i: (0,)),
            pl.BlockSpec((H, O), lambda i: (0, 0)),
            pl.BlockSpec((O,), lambda i: (0,)),
        ],
        out_specs=pl.BlockSpec((BR, O), lambda i: (i, 0)),
        out_shape=jax.ShapeDtypeStruct((NACC, O), jnp.float32),
    )(acc, den, bias, W1, b1, W2, b2)


def kernel(x, edge_index, W, att_src, att_dst, bias, W1, b1, W2, b2):
    h, a_s2, a_d2, cvec = _tc_head(x, W, att_src, att_dst)

    loop = jnp.arange(N, dtype=jnp.int32)
    npad = EPAD - E - N
    src = jnp.concatenate([edge_index[0], loop,
                           jnp.zeros((npad,), jnp.int32)])
    dst = jnp.concatenate([edge_index[1], loop,
                           jnp.full((npad,), DUMMY, jnp.int32)])
    src = src.reshape(TOT_CHUNKS, C)
    dst = dst.reshape(TOT_CHUNKS, C)

    a_s = jnp.pad(a_s2.reshape(-1), (0, PADN - N))
    a_d = jnp.pad(a_d2.reshape(-1), (0, PADN - N))

    acc, den = _sc_gat(h, src, dst, a_s, a_d, cvec.reshape(-1))
    y = _tc_mlp(acc, den, bias, W1, b1, W2, b2)
    return y[:N]


# speedup vs baseline: 21.5164x; 1.3252x over previous
import dataclasses
import functools

import jax
import jax.numpy as jnp
from jax import lax
from jax.experimental import pallas as pl
from jax.experimental.pallas import tpu as pltpu
from jax.experimental.pallas import tpu_sc as plsc

N = 10000
E = 320000
D = 128
H = 256
O = 64

NLANE = 16
NCORE = 2
NSUB = 16
NW = NCORE * NSUB
C = 64
K = 164
EPAD = NW * K * C
TOT_CHUNKS = EPAD // C
NACC = 10240
ROWS_PER_TILE = NACC // NSUB
PADN = 10016
DUMMY = N


def _tc_head(x, W, att_src, att_dst):
    def body(x_ref, w_ref, as_ref, ad_ref, h_ref, s_ref, d_ref, c_ref):
        h = jnp.dot(x_ref[...], w_ref[...], preferred_element_type=jnp.float32)
        h_ref[...] = h
        a_s = jnp.sum(h * as_ref[...][None, :], axis=1, keepdims=True)
        a_d = jnp.sum(h * ad_ref[...][None, :], axis=1, keepdims=True)
        s_ref[...] = a_s
        d_ref[...] = a_d
        c = jnp.maximum(jnp.max(a_s) + jnp.max(a_d), 0.0)
        c_ref[...] = jnp.full((1, NLANE), c, jnp.float32)

    return pl.pallas_call(
        body,
        out_shape=(
            jax.ShapeDtypeStruct((N, D), jnp.float32),
            jax.ShapeDtypeStruct((N, 1), jnp.float32),
            jax.ShapeDtypeStruct((N, 1), jnp.float32),
            jax.ShapeDtypeStruct((1, NLANE), jnp.float32),
        ),
    )(x, W, att_src, att_dst)


def _sc_gat(h, src_idx, dst_idx, a_s, a_d, cvec):
    mesh = plsc.VectorSubcoreMesh(core_axis_name="c", subcore_axis_name="s")
    cp = pltpu.CompilerParams()
    if "needs_layout_passes" in pltpu.CompilerParams.__dataclass_fields__:
        cp = dataclasses.replace(cp, needs_layout_passes=False)

    @functools.partial(
        pl.kernel,
        compiler_params=cp,
        out_type=(
            jax.ShapeDtypeStruct((NCORE, NACC, D), jnp.float32),
            jax.ShapeDtypeStruct((NW, NACC), jnp.float32),
        ),
        mesh=mesh,
        scratch_types=[
            pltpu.VMEM((PADN,), jnp.float32),
            pltpu.VMEM((PADN,), jnp.float32),
            pltpu.VMEM((NLANE,), jnp.float32),
            pltpu.VMEM((C,), jnp.int32),
            pltpu.VMEM((C,), jnp.int32),
            pltpu.VMEM((C,), jnp.int32),
            pltpu.VMEM((C,), jnp.int32),
            pltpu.VMEM((C,), jnp.int32),
            pltpu.VMEM((C,), jnp.int32),
            pltpu.VMEM((C, D), jnp.float32),
            pltpu.VMEM((C, D), jnp.float32),
            pltpu.VMEM((C,), jnp.float32),
            pltpu.VMEM((C,), jnp.float32),
            pltpu.VMEM((NACC,), jnp.float32),
            pltpu.VMEM_SHARED((NACC, D), jnp.float32),
            pltpu.SemaphoreType.DMA,
            pltpu.SemaphoreType.DMA,
            pltpu.SemaphoreType.DMA,
            pltpu.SemaphoreType.DMA,
            pltpu.SemaphoreType.DMA,
            pltpu.SemaphoreType.DMA,
        ],
    )
    def kern(h_hbm, src_hbm, dst_hbm, as_hbm, ad_hbm, c_hbm, acc_out, den_out,
             as_v, ad_v, c_v, srcA, dstA, dsA, srcB, dstB, dsB,
             rowsA, rowsB, pA, pB, den_v, acc_sh,
             semiA, semiB, semgA, semgB, semsA, semsB):
        cid = lax.axis_index("c")
        sid = lax.axis_index("s")
        wid = sid * NCORE + cid

        pltpu.sync_copy(as_hbm, as_v)
        pltpu.sync_copy(ad_hbm, ad_v)
        pltpu.sync_copy(c_hbm, c_v)
        shift = c_v[...]

        slots = ((srcA, dstA, dsA, rowsA, pA, semiA, semgA, semsA),
                 (srcB, dstB, dsB, rowsB, pB, semiB, semgB, semsB))

        def issue_idx(slot, ck):
            src_v, dst_v = slots[slot][0], slots[slot][1]
            semi = slots[slot][5]
            pltpu.async_copy(src_hbm.at[ck], src_v, semi)
            pltpu.async_copy(dst_hbm.at[ck], dst_v, semi)

        def wait_idx(slot):
            src_v, dst_v = slots[slot][0], slots[slot][1]
            semi = slots[slot][5]
            pltpu.make_async_copy(src_hbm.at[0], src_v, semi).wait()
            pltpu.make_async_copy(dst_hbm.at[0], dst_v, semi).wait()

        def issue_gather(slot):
            src_v, rows_v, semg = slots[slot][0], slots[slot][3], slots[slot][6]
            pltpu.async_copy(h_hbm.at[src_v], rows_v, semg)

        def wait_gather(slot):
            rows_v, semg = slots[slot][3], slots[slot][6]
            pltpu.make_async_copy(h_hbm.at[pl.ds(0, C)], rows_v, semg).wait()

        def issue_scatter(slot):
            ds_v, rows_v, sems = slots[slot][2], slots[slot][3], slots[slot][7]
            pltpu.async_copy(rows_v, acc_sh.at[ds_v], sems, add=True)

        def wait_scatter(slot):
            rows_v, sems = slots[slot][3], slots[slot][7]
            pltpu.make_async_copy(rows_v, acc_sh.at[pl.ds(0, C)], sems).wait()

        def compute_p(slot):
            src_v, dst_v, ds_v, _, p_v = slots[slot][:5]
            for b in range(C // NLANE):
                sl = pl.ds(b * NLANE, NLANE)
                d16 = dst_v[sl]
                ds_v[sl] = d16
                av = plsc.load_gather(as_v, [src_v[sl]])
                bv = plsc.load_gather(ad_v, [d16])
                e = av + bv
                e = jnp.maximum(e, e * 0.2)
                p16 = jnp.exp(e - shift)
                p_v[sl] = p16
                plsc.addupdate_scatter(den_v, [d16], p16)

        def scale(slot):
            rows_v, p_v = slots[slot][3], slots[slot][4]

            @pl.loop(0, C)
            def _(j):
                jv = jnp.full((NLANE,), j, jnp.int32)
                pvec = plsc.load_gather(p_v, [jv])
                for q in range(D // NLANE):
                    sl = pl.ds(q * NLANE, NLANE)
                    rows_v[j, sl] = rows_v[j, sl] * pvec

        zv = jnp.zeros((NLANE,), jnp.float32)

        @pl.loop(0, NACC, step=NLANE)
        def _(i):
            den_v[pl.ds(i, NLANE)] = zv

        @pl.loop(0, C)
        def _(j):
            for q in range(D // NLANE):
                rowsA[j, pl.ds(q * NLANE, NLANE)] = zv

        zbase = sid * ROWS_PER_TILE
        for t in range(ROWS_PER_TILE // C):
            pltpu.sync_copy(rowsA, acc_sh.at[pl.ds(zbase + t * C, C)])

        chunk0 = wid * K
        issue_idx(0, chunk0)
        issue_idx(1, chunk0 + 1)
        wait_idx(0)
        issue_gather(0)
        wait_idx(1)
        issue_gather(1)
        plsc.subcore_barrier()

        @pl.loop(0, K // 2 - 1)
        def _(i):
            k0 = chunk0 + 2 * i
            for s in (0, 1):
                compute_p(s)
                wait_gather(s)
                issue_idx(s, k0 + s + 2)
                scale(s)
                issue_scatter(s)
            for s in (0, 1):
                wait_scatter(s)
                wait_idx(s)
                issue_gather(s)

        for s in (0, 1):
            compute_p(s)
            wait_gather(s)
            scale(s)
            issue_scatter(s)
        wait_scatter(0)
        wait_scatter(1)

        plsc.subcore_barrier()
        pltpu.sync_copy(acc_sh.at[pl.ds(zbase, ROWS_PER_TILE)],
                        acc_out.at[cid, pl.ds(zbase, ROWS_PER_TILE)])
        pltpu.sync_copy(den_v, den_out.at[wid])

    return kern(h, src_idx, dst_idx, a_s, a_d, cvec)


def _tc_mlp(acc, den, bias, W1, b1, W2, b2):
    BR = 1024

    def body(acc_ref, den_ref, bias_ref, w1_ref, b1_ref, w2_ref, b2_ref,
             y_ref):
        g = acc_ref[0] + acc_ref[1]
        d = jnp.sum(den_ref[...], axis=0).reshape(BR, 1)
        gat = g / d + bias_ref[...][None, :]
        z = jnp.dot(gat, w1_ref[...], preferred_element_type=jnp.float32)
        z = jnp.maximum(z + b1_ref[...][None, :], 0.0)
        y = jnp.dot(z, w2_ref[...], preferred_element_type=jnp.float32)
        y_ref[...] = jax.nn.sigmoid(y + b2_ref[...][None, :])

    return pl.pallas_call(
        body,
        grid=(NACC // BR,),
        in_specs=[
            pl.BlockSpec((NCORE, BR, D), lambda i: (0, i, 0)),
            pl.BlockSpec((NW, BR), lambda i: (0, i)),
            pl.BlockSpec((D,), lambda i: (0,)),
            pl.BlockSpec((D, H), lambda i: (0, 0)),
            pl.BlockSpec((H,), lambda i: (0,)),
            pl.BlockSpec((H, O), lambda i: (0, 0)),
            pl.BlockSpec((O,), lambda i: (0,)),
        ],
        out_specs=pl.BlockSpec((BR, O), lambda i: (i, 0)),
        out_shape=jax.ShapeDtypeStruct((NACC, O), jnp.float32),
    )(acc, den, bias, W1, b1, W2, b2)


def kernel(x, edge_index, W, att_src, att_dst, bias, W1, b1, W2, b2):
    h, a_s2, a_d2, cvec = _tc_head(x, W, att_src, att_dst)

    loop = jnp.arange(N, dtype=jnp.int32)
    npad = EPAD - E - N
    src = jnp.concatenate([edge_index[0], loop,
                           jnp.zeros((npad,), jnp.int32)])
    dst = jnp.concatenate([edge_index[1], loop,
                           jnp.full((npad,), DUMMY, jnp.int32)])
    src = src.reshape(TOT_CHUNKS, C)
    dst = dst.reshape(TOT_CHUNKS, C)

    a_s = jnp.pad(a_s2.reshape(-1), (0, PADN - N))
    a_d = jnp.pad(a_d2.reshape(-1), (0, PADN - N))

    acc, den = _sc_gat(h, src, dst, a_s, a_d, cvec.reshape(-1))
    y = _tc_mlp(acc, den, bias, W1, b1, W2, b2)
    return y[:N]
